# baseline (speedup 1.0000x reference)
import functools

import jax
import jax.numpy as jnp
from jax import lax
from jax.experimental import pallas as pl
from jax.experimental.pallas import tpu as pltpu
from jax.experimental.pallas import tpu_sc as plsc

_SCALING = 16 / 16

_R = 16
_V = 100000
_D = 128
_B = 4096
_L = 200
_NTOK = _B * _L

_NC = 2
_NS = 16
_NW = _NC * _NS
_RPC = 128
_NCHUNK = _NTOK // (_NW * _RPC)
_NBUF = 4

_sc_mesh = plsc.VectorSubcoreMesh(core_axis_name="c", subcore_axis_name="s")


@functools.partial(
    pl.kernel,
    out_type=jax.ShapeDtypeStruct((_NW, _NCHUNK, _RPC, _R), jnp.float32),
    mesh=_sc_mesh,
    scratch_types=[
        pltpu.VMEM((_NCHUNK, _RPC), jnp.int32),
        [pltpu.VMEM((_RPC, _R), jnp.float32) for _ in range(_NBUF)],
        [pltpu.SemaphoreType.DMA for _ in range(_NBUF)],
    ],
    compiler_params=pltpu.CompilerParams(use_tc_tiling_on_sc=False),
)
def _gather(table_hbm, idx_hbm, out_hbm, idx_v, bufs, sems):
    wid = lax.axis_index("s") * _NC + lax.axis_index("c")
    pltpu.sync_copy(idx_hbm.at[wid], idx_v)

    for s in range(_NBUF):
        pltpu.async_copy(table_hbm.at[idx_v.at[s]], bufs[s], sems[s])

    def body(jj, carry):
        j0 = jj * _NBUF
        for s in range(_NBUF):
            j = j0 + s
            pltpu.make_async_copy(
                table_hbm.at[pl.ds(0, _RPC)], bufs[s], sems[s]).wait()
            pltpu.sync_copy(bufs[s], out_hbm.at[wid, j])
            jn = jnp.minimum(j + _NBUF, _NCHUNK - 1)
            pltpu.async_copy(table_hbm.at[idx_v.at[jn]], bufs[s], sems[s])
        return carry

    lax.fori_loop(0, _NCHUNK // _NBUF, body, 0)

    for s in range(_NBUF):
        pltpu.make_async_copy(
            table_hbm.at[pl.ds(0, _RPC)], bufs[s], sems[s]).wait()


_PACK = 8
_NROW = _NTOK // _PACK
_RBLK = 2048


def _proj_body(emb_ref, bt_ref, o_ref, w_ref):
    @pl.when(pl.program_id(0) == 0)
    def _():
        w_ref[...] = jnp.zeros((_PACK * _R, _PACK * _D), jnp.float32)
        for i in range(_PACK):
            w_ref[i * _R:(i + 1) * _R, i * _D:(i + 1) * _D] = (
                bt_ref[...] * _SCALING)

    o_ref[...] = lax.dot_general(
        emb_ref[...], w_ref[...],
        dimension_numbers=(((1,), (0,)), ((), ())),
        preferred_element_type=jnp.float32,
    )


def _project(emb_packed, lora_Bt):
    return pl.pallas_call(
        _proj_body,
        grid=(_NROW // _RBLK,),
        in_specs=[
            pl.BlockSpec((_RBLK, _PACK * _R), lambda i: (i, 0)),
            pl.BlockSpec((_R, _D), lambda i: (0, 0)),
        ],
        out_specs=pl.BlockSpec((_RBLK, _PACK * _D), lambda i: (i, 0)),
        out_shape=jax.ShapeDtypeStruct((_NROW, _PACK * _D), jnp.float32),
        scratch_shapes=[pltpu.VMEM((_PACK * _R, _PACK * _D), jnp.float32)],
    )(emb_packed, lora_Bt)


def kernel(x, lora_A, lora_B):
    idx = x.astype(jnp.int32).reshape(_NW, _NCHUNK, _RPC)
    emb = _gather(lora_A.T, idx)
    emb_packed = emb.reshape(_NROW, _PACK * _R)
    out = _project(emb_packed, lora_B.T)
    return out.reshape(_B, _L, _D)


# speedup vs baseline: 1.8015x; 1.8015x over previous
import functools

import jax
import jax.numpy as jnp
from jax import lax
from jax.experimental import pallas as pl
from jax.experimental.pallas import tpu as pltpu
from jax.experimental.pallas import tpu_sc as plsc

_SCALING = 16 / 16

_R = 16
_V = 100000
_D = 128
_B = 4096
_L = 200
_NTOK = _B * _L

_NC = 2
_NS = 16
_NW = _NC * _NS
_RPC = 128
_NCHUNK = _NTOK // (_NW * _RPC)

_VBLK = 2000


def _table_body(a_ref, b_ref, o_ref):
    o_ref[...] = lax.dot_general(
        a_ref[...], b_ref[...],
        dimension_numbers=(((1,), (1,)), ((), ())),
        preferred_element_type=jnp.float32,
    ) * _SCALING


def _build_table(lora_At, lora_B):
    return pl.pallas_call(
        _table_body,
        grid=(_V // _VBLK,),
        in_specs=[
            pl.BlockSpec((_VBLK, _R), lambda i: (i, 0)),
            pl.BlockSpec((_D, _R), lambda i: (0, 0)),
        ],
        out_specs=pl.BlockSpec((_VBLK, _D), lambda i: (i, 0)),
        out_shape=jax.ShapeDtypeStruct((_V, _D), jnp.float32),
    )(lora_At, lora_B)


_sc_mesh = plsc.VectorSubcoreMesh(core_axis_name="c", subcore_axis_name="s")


_NBUF = 4


@functools.partial(
    pl.kernel,
    out_type=jax.ShapeDtypeStruct((_NW, _NCHUNK, _RPC, _D), jnp.float32),
    mesh=_sc_mesh,
    scratch_types=[
        pltpu.VMEM((_NCHUNK, _RPC), jnp.int32),
        [pltpu.VMEM((_RPC, _D), jnp.float32) for _ in range(_NBUF)],
        [pltpu.SemaphoreType.DMA for _ in range(_NBUF)],
    ],
)
def _gather(table_hbm, idx_hbm, out_hbm, idx_v, bufs, sems):
    wid = lax.axis_index("s") * _NC + lax.axis_index("c")
    pltpu.sync_copy(idx_hbm.at[wid], idx_v)

    for s in range(_NBUF):
        pltpu.async_copy(table_hbm.at[idx_v.at[s]], bufs[s], sems[s])

    def body(jj, carry):
        j0 = jj * _NBUF
        for s in range(_NBUF):
            j = j0 + s
            pltpu.make_async_copy(
                table_hbm.at[pl.ds(0, _RPC)], bufs[s], sems[s]).wait()
            pltpu.sync_copy(bufs[s], out_hbm.at[wid, j])
            jn = jnp.minimum(j + _NBUF, _NCHUNK - 1)
            pltpu.async_copy(table_hbm.at[idx_v.at[jn]], bufs[s], sems[s])
        return carry

    lax.fori_loop(0, _NCHUNK // _NBUF, body, 0)

    for s in range(_NBUF):
        pltpu.make_async_copy(
            table_hbm.at[pl.ds(0, _RPC)], bufs[s], sems[s]).wait()


def kernel(x, lora_A, lora_B):
    idx = x.astype(jnp.int32).reshape(_NW, _NCHUNK, _RPC)
    table = _build_table(lora_A.T, lora_B)
    out = _gather(table, idx)
    return out.reshape(_B, _L, _D)


# speedup vs baseline: 1.8173x; 1.0088x over previous
import functools

import jax
import jax.numpy as jnp
from jax import lax
from jax.experimental import pallas as pl
from jax.experimental.pallas import tpu as pltpu
from jax.experimental.pallas import tpu_sc as plsc

_SCALING = 16 / 16

_R = 16
_V = 100000
_D = 128
_B = 4096
_L = 200
_NTOK = _B * _L

_NC = 2
_NS = 16
_NW = _NC * _NS
_RPC = 128
_NCHUNK = _NTOK // (_NW * _RPC)

_VBLK = 2000


def _table_body(a_ref, b_ref, o_ref):
    o_ref[...] = lax.dot_general(
        a_ref[...], b_ref[...],
        dimension_numbers=(((1,), (1,)), ((), ())),
        preferred_element_type=jnp.float32,
    ) * _SCALING


def _build_table(lora_At, lora_B):
    return pl.pallas_call(
        _table_body,
        grid=(_V // _VBLK,),
        in_specs=[
            pl.BlockSpec((_VBLK, _R), lambda i: (i, 0)),
            pl.BlockSpec((_D, _R), lambda i: (0, 0)),
        ],
        out_specs=pl.BlockSpec((_VBLK, _D), lambda i: (i, 0)),
        out_shape=jax.ShapeDtypeStruct((_V, _D), jnp.float32),
    )(lora_At, lora_B)


_sc_mesh = plsc.VectorSubcoreMesh(core_axis_name="c", subcore_axis_name="s")


_NBUF = 4
_LAG = 2


def _wait_gather(table_hbm, buf, sem):
    pltpu.make_async_copy(table_hbm.at[pl.ds(0, _RPC)], buf, sem).wait()


def _wait_write(out_hbm, buf, sem):
    pltpu.make_async_copy(buf, out_hbm.at[0, 0], sem).wait()


@functools.partial(
    pl.kernel,
    out_type=jax.ShapeDtypeStruct((_NW, _NCHUNK, _RPC, _D), jnp.float32),
    mesh=_sc_mesh,
    scratch_types=[
        pltpu.VMEM((_NCHUNK, _RPC), jnp.int32),
        [pltpu.VMEM((_RPC, _D), jnp.float32) for _ in range(_NBUF)],
        [pltpu.SemaphoreType.DMA for _ in range(_NBUF)],
        [pltpu.SemaphoreType.DMA for _ in range(_NBUF)],
    ],
)
def _gather(table_hbm, idx_hbm, out_hbm, idx_v, bufs, gsems, wsems):
    wid = lax.axis_index("s") * _NC + lax.axis_index("c")
    pltpu.sync_copy(idx_hbm.at[wid], idx_v)

    for s in range(_NBUF):
        pltpu.async_copy(table_hbm.at[idx_v.at[s]], bufs[s], gsems[s])

    for s in range(_NBUF):
        _wait_gather(table_hbm, bufs[s], gsems[s])
        pltpu.async_copy(bufs[s], out_hbm.at[wid, s], wsems[s])
        if s >= _LAG:
            s2 = s - _LAG
            _wait_write(out_hbm, bufs[s2], wsems[s2])
            pltpu.async_copy(
                table_hbm.at[idx_v.at[s + _NBUF - _LAG]], bufs[s2], gsems[s2])

    def body(jj, carry):
        j0 = jj * _NBUF
        for s in range(_NBUF):
            j = j0 + s
            _wait_gather(table_hbm, bufs[s], gsems[s])
            pltpu.async_copy(bufs[s], out_hbm.at[wid, j], wsems[s])
            s2 = (s - _LAG) % _NBUF
            jr = jnp.minimum(j + _NBUF - _LAG, _NCHUNK - 1)
            _wait_write(out_hbm, bufs[s2], wsems[s2])
            pltpu.async_copy(table_hbm.at[idx_v.at[jr]], bufs[s2], gsems[s2])
        return carry

    lax.fori_loop(1, _NCHUNK // _NBUF, body, 0)

    for i in range(_NCHUNK - _LAG, _NCHUNK):
        _wait_gather(table_hbm, bufs[(i - _LAG) % _NBUF],
                     gsems[(i - _LAG) % _NBUF])
        _wait_write(out_hbm, bufs[i % _NBUF], wsems[i % _NBUF])


def kernel(x, lora_A, lora_B):
    idx = x.astype(jnp.int32).reshape(_NW, _NCHUNK, _RPC)
    table = _build_table(lora_A.T, lora_B)
    out = _gather(table, idx)
    return out.reshape(_B, _L, _D)


# speedup vs baseline: 1.8685x; 1.0282x over previous
import functools

import jax
import jax.numpy as jnp
from jax import lax
from jax.experimental import pallas as pl
from jax.experimental.pallas import tpu as pltpu
from jax.experimental.pallas import tpu_sc as plsc

_SCALING = 16 / 16

_R = 16
_V = 100000
_D = 128
_B = 4096
_L = 200
_NTOK = _B * _L

_NC = 2
_NS = 16
_NW = _NC * _NS
_RPC = 128
_NCHUNK = _NTOK // (_NW * _RPC)

_VBLK = 2048


def _table_body(a_ref, b_ref, o_ref):
    o_ref[...] = lax.dot_general(
        a_ref[...], b_ref[...],
        dimension_numbers=(((0,), (1,)), ((), ())),
        preferred_element_type=jnp.float32,
    ) * _SCALING


def _build_table(lora_A, lora_B):
    return pl.pallas_call(
        _table_body,
        grid=(pl.cdiv(_V, _VBLK),),
        in_specs=[
            pl.BlockSpec((_R, _VBLK), lambda i: (0, i)),
            pl.BlockSpec((_D, _R), lambda i: (0, 0)),
        ],
        out_specs=pl.BlockSpec((_VBLK, _D), lambda i: (i, 0)),
        out_shape=jax.ShapeDtypeStruct((_V, _D), jnp.float32),
    )(lora_A, lora_B)


_sc_mesh = plsc.VectorSubcoreMesh(core_axis_name="c", subcore_axis_name="s")


_NBUF = 4
_LAG = 2


def _wait_gather(table_hbm, buf, sem):
    pltpu.make_async_copy(table_hbm.at[pl.ds(0, _RPC)], buf, sem).wait()


def _wait_write(out_hbm, buf, sem):
    pltpu.make_async_copy(buf, out_hbm.at[0, 0], sem).wait()


@functools.partial(
    pl.kernel,
    out_type=jax.ShapeDtypeStruct((_NW, _NCHUNK, _RPC, _D), jnp.float32),
    mesh=_sc_mesh,
    scratch_types=[
        pltpu.VMEM((_NCHUNK, _RPC), jnp.int32),
        [pltpu.VMEM((_RPC, _D), jnp.float32) for _ in range(_NBUF)],
        [pltpu.SemaphoreType.DMA for _ in range(_NBUF)],
        [pltpu.SemaphoreType.DMA for _ in range(_NBUF)],
    ],
)
def _gather(table_hbm, idx_hbm, out_hbm, idx_v, bufs, gsems, wsems):
    wid = lax.axis_index("s") * _NC + lax.axis_index("c")
    pltpu.sync_copy(idx_hbm.at[wid], idx_v)

    for s in range(_NBUF):
        pltpu.async_copy(table_hbm.at[idx_v.at[s]], bufs[s], gsems[s])

    for s in range(_NBUF):
        _wait_gather(table_hbm, bufs[s], gsems[s])
        pltpu.async_copy(bufs[s], out_hbm.at[wid, s], wsems[s])
        if s >= _LAG:
            s2 = s - _LAG
            _wait_write(out_hbm, bufs[s2], wsems[s2])
            pltpu.async_copy(
                table_hbm.at[idx_v.at[s + _NBUF - _LAG]], bufs[s2], gsems[s2])

    def body(jj, carry):
        j0 = jj * _NBUF
        for s in range(_NBUF):
            j = j0 + s
            _wait_gather(table_hbm, bufs[s], gsems[s])
            pltpu.async_copy(bufs[s], out_hbm.at[wid, j], wsems[s])
            s2 = (s - _LAG) % _NBUF
            jr = jnp.minimum(j + _NBUF - _LAG, _NCHUNK - 1)
            _wait_write(out_hbm, bufs[s2], wsems[s2])
            pltpu.async_copy(table_hbm.at[idx_v.at[jr]], bufs[s2], gsems[s2])
        return carry

    lax.fori_loop(1, _NCHUNK // _NBUF, body, 0)

    for i in range(_NCHUNK - _LAG, _NCHUNK):
        _wait_gather(table_hbm, bufs[(i - _LAG) % _NBUF],
                     gsems[(i - _LAG) % _NBUF])
        _wait_write(out_hbm, bufs[i % _NBUF], wsems[i % _NBUF])


def kernel(x, lora_A, lora_B):
    idx = x.astype(jnp.int32).reshape(_NW, _NCHUNK, _RPC)
    table = _build_table(lora_A, lora_B)
    out = _gather(table, idx)
    return out.reshape(_B, _L, _D)


# speedup vs baseline: 1.9703x; 1.0544x over previous
import functools

import jax
import jax.numpy as jnp
from jax import lax
from jax.experimental import pallas as pl
from jax.experimental.pallas import tpu as pltpu
from jax.experimental.pallas import tpu_sc as plsc

_SCALING = 16 / 16

_R = 16
_V = 100000
_D = 128
_B = 4096
_L = 200
_NTOK = _B * _L

_NC = 2
_NS = 16
_NW = _NC * _NS
_RPC = 128
_NCHUNK = _NTOK // (_NW * _RPC)

_VBLK = 8192


def _table_body(a_ref, b_ref, o_ref):
    o_ref[...] = lax.dot_general(
        a_ref[...], b_ref[...],
        dimension_numbers=(((0,), (1,)), ((), ())),
        preferred_element_type=jnp.float32,
    ) * _SCALING


def _build_table(lora_A, lora_B):
    return pl.pallas_call(
        _table_body,
        grid=(pl.cdiv(_V, _VBLK),),
        in_specs=[
            pl.BlockSpec((_R, _VBLK), lambda i: (0, i)),
            pl.BlockSpec((_D, _R), lambda i: (0, 0)),
        ],
        out_specs=pl.BlockSpec((_VBLK, _D), lambda i: (i, 0)),
        out_shape=jax.ShapeDtypeStruct((_V, _D), jnp.float32),
    )(lora_A, lora_B)


_sc_mesh = plsc.VectorSubcoreMesh(core_axis_name="c", subcore_axis_name="s")


_NBUF = 4
_LAG = 2


def _wait_gather(table_hbm, buf, sem):
    pltpu.make_async_copy(table_hbm.at[pl.ds(0, _RPC)], buf, sem).wait()


def _wait_write(out_hbm, buf, sem):
    pltpu.make_async_copy(buf, out_hbm.at[0, 0], sem).wait()


@functools.partial(
    pl.kernel,
    out_type=jax.ShapeDtypeStruct((_NW, _NCHUNK, _RPC, _D), jnp.float32),
    mesh=_sc_mesh,
    scratch_types=[
        pltpu.VMEM((_NCHUNK, _RPC), jnp.int32),
        [pltpu.VMEM((_RPC, _D), jnp.float32) for _ in range(_NBUF)],
        [pltpu.SemaphoreType.DMA for _ in range(_NBUF)],
        [pltpu.SemaphoreType.DMA for _ in range(_NBUF)],
    ],
)
def _gather(table_hbm, idx_hbm, out_hbm, idx_v, bufs, gsems, wsems):
    wid = lax.axis_index("s") * _NC + lax.axis_index("c")
    pltpu.sync_copy(idx_hbm.at[wid], idx_v)

    for s in range(_NBUF):
        pltpu.async_copy(table_hbm.at[idx_v.at[s]], bufs[s], gsems[s])

    for s in range(_NBUF):
        _wait_gather(table_hbm, bufs[s], gsems[s])
        pltpu.async_copy(bufs[s], out_hbm.at[wid, s], wsems[s])
        if s >= _LAG:
            s2 = s - _LAG
            _wait_write(out_hbm, bufs[s2], wsems[s2])
            pltpu.async_copy(
                table_hbm.at[idx_v.at[s + _NBUF - _LAG]], bufs[s2], gsems[s2])

    def body(jj, carry):
        j0 = jj * _NBUF
        for s in range(_NBUF):
            j = j0 + s
            _wait_gather(table_hbm, bufs[s], gsems[s])
            pltpu.async_copy(bufs[s], out_hbm.at[wid, j], wsems[s])
            s2 = (s - _LAG) % _NBUF
            jr = jnp.minimum(j + _NBUF - _LAG, _NCHUNK - 1)
            _wait_write(out_hbm, bufs[s2], wsems[s2])
            pltpu.async_copy(table_hbm.at[idx_v.at[jr]], bufs[s2], gsems[s2])
        return carry

    lax.fori_loop(1, _NCHUNK // _NBUF, body, 0)

    for i in range(_NCHUNK - _LAG, _NCHUNK):
        _wait_gather(table_hbm, bufs[(i - _LAG) % _NBUF],
                     gsems[(i - _LAG) % _NBUF])
        _wait_write(out_hbm, bufs[i % _NBUF], wsems[i % _NBUF])


def kernel(x, lora_A, lora_B):
    idx = x.astype(jnp.int32).reshape(_NW, _NCHUNK, _RPC)
    table = _build_table(lora_A, lora_B)
    out = _gather(table, idx)
    return out.reshape(_B, _L, _D)


# speedup vs baseline: 1.9752x; 1.0025x over previous
import functools

import jax
import jax.numpy as jnp
from jax import lax
from jax.experimental import pallas as pl
from jax.experimental.pallas import tpu as pltpu
from jax.experimental.pallas import tpu_sc as plsc

_SCALING = 16 / 16

_R = 16
_V = 100000
_D = 128
_B = 4096
_L = 200
_NTOK = _B * _L

_NC = 2
_NS = 16
_NW = _NC * _NS
_RPC = 128
_NCHUNK = _NTOK // (_NW * _RPC)

_VBLK = 8192


def _table_body(a_ref, b_ref, o_ref):
    o_ref[...] = lax.dot_general(
        a_ref[...], b_ref[...],
        dimension_numbers=(((0,), (1,)), ((), ())),
        preferred_element_type=jnp.float32,
    ) * _SCALING


def _build_table(lora_A, lora_B):
    return pl.pallas_call(
        _table_body,
        grid=(pl.cdiv(_V, _VBLK),),
        in_specs=[
            pl.BlockSpec((_R, _VBLK), lambda i: (0, i)),
            pl.BlockSpec((_D, _R), lambda i: (0, 0)),
        ],
        out_specs=pl.BlockSpec((_VBLK, _D), lambda i: (i, 0)),
        out_shape=jax.ShapeDtypeStruct((_V, _D), jnp.float32),
    )(lora_A, lora_B)


_sc_mesh = plsc.VectorSubcoreMesh(core_axis_name="c", subcore_axis_name="s")


_PAIR = 2
_NPAIR = _NCHUNK // _PAIR
_NSLOT = 3


def _wait_gather(table_hbm, buf, sem):
    pltpu.make_async_copy(table_hbm.at[pl.ds(0, _RPC)], buf, sem).wait()


def _wait_write(out_hbm, buf, sem):
    pltpu.make_async_copy(buf, out_hbm.at[0, pl.ds(0, _PAIR)], sem).wait()


@functools.partial(
    pl.kernel,
    out_type=jax.ShapeDtypeStruct((_NW, _NCHUNK, _RPC, _D), jnp.float32),
    mesh=_sc_mesh,
    scratch_types=[
        pltpu.VMEM((_NCHUNK, _RPC), jnp.int32),
        [pltpu.VMEM((_PAIR, _RPC, _D), jnp.float32) for _ in range(_NSLOT)],
        [[pltpu.SemaphoreType.DMA for _ in range(_PAIR)]
         for _ in range(_NSLOT)],
        [pltpu.SemaphoreType.DMA for _ in range(_NSLOT)],
    ],
)
def _gather(table_hbm, idx_hbm, out_hbm, idx_v, bufs, gsems, wsems):
    wid = lax.axis_index("s") * _NC + lax.axis_index("c")
    pltpu.sync_copy(idx_hbm.at[wid], idx_v)

    def fill(sl, q):
        for k in range(_PAIR):
            pltpu.async_copy(table_hbm.at[idx_v.at[q * _PAIR + k]],
                             bufs[sl].at[k], gsems[sl][k])

    for sl in range(_NSLOT - 1):
        fill(sl, sl)

    def step(q, sl, refill):
        for k in range(_PAIR):
            _wait_gather(table_hbm, bufs[sl].at[k], gsems[sl][k])
        pltpu.async_copy(bufs[sl], out_hbm.at[wid, pl.ds(q * _PAIR, _PAIR)],
                         wsems[sl])
        if refill:
            sl2 = (sl - 1) % _NSLOT
            _wait_write(out_hbm, bufs[sl2], wsems[sl2])
            qr = jnp.minimum(q + _NSLOT - 1, _NPAIR - 1)
            for k in range(_PAIR):
                pltpu.async_copy(table_hbm.at[idx_v.at[qr * _PAIR + k]],
                                 bufs[sl2].at[k], gsems[sl2][k])

    step(0, 0, refill=False)
    fill(_NSLOT - 1, _NSLOT - 1)

    def body(bb, carry):
        q0 = 1 + bb * _NSLOT
        for u in range(_NSLOT):
            step(q0 + u, (1 + u) % _NSLOT, refill=True)
        return carry

    lax.fori_loop(0, (_NPAIR - 1) // _NSLOT, body, 0)

    _wait_write(out_hbm, bufs[(_NPAIR - 1) % _NSLOT],
                wsems[(_NPAIR - 1) % _NSLOT])
    for q in (_NPAIR - 2, _NPAIR - 1):
        sl2 = (q - 1) % _NSLOT
        for k in range(_PAIR):
            _wait_gather(table_hbm, bufs[sl2].at[k], gsems[sl2][k])


def kernel(x, lora_A, lora_B):
    idx = x.astype(jnp.int32).reshape(_NW, _NCHUNK, _RPC)
    table = _build_table(lora_A, lora_B)
    out = _gather(table, idx)
    return out.reshape(_B, _L, _D)


# speedup vs baseline: 1.9819x; 1.0034x over previous
import functools

import jax
import jax.numpy as jnp
from jax import lax
from jax.experimental import pallas as pl
from jax.experimental.pallas import tpu as pltpu
from jax.experimental.pallas import tpu_sc as plsc

_SCALING = 16 / 16

_R = 16
_V = 100000
_D = 128
_B = 4096
_L = 200
_NTOK = _B * _L

_NC = 2
_NS = 16
_NW = _NC * _NS
_RPC = 128
_NCHUNK = _NTOK // (_NW * _RPC)

_VBLK = 16384


def _table_body(a_ref, b_ref, o_ref):
    o_ref[...] = lax.dot_general(
        a_ref[...], b_ref[...],
        dimension_numbers=(((0,), (1,)), ((), ())),
        preferred_element_type=jnp.float32,
    ) * _SCALING


def _build_table(lora_A, lora_B):
    return pl.pallas_call(
        _table_body,
        grid=(pl.cdiv(_V, _VBLK),),
        in_specs=[
            pl.BlockSpec((_R, _VBLK), lambda i: (0, i)),
            pl.BlockSpec((_D, _R), lambda i: (0, 0)),
        ],
        out_specs=pl.BlockSpec((_VBLK, _D), lambda i: (i, 0)),
        out_shape=jax.ShapeDtypeStruct((_V, _D), jnp.float32),
    )(lora_A, lora_B)


_sc_mesh = plsc.VectorSubcoreMesh(core_axis_name="c", subcore_axis_name="s")


_PAIR = 2
_NPAIR = _NCHUNK // _PAIR
_NSLOT = 3


def _wait_gather(table_hbm, buf, sem):
    pltpu.make_async_copy(table_hbm.at[pl.ds(0, _RPC)], buf, sem).wait()


def _wait_write(out_hbm, buf, sem):
    pltpu.make_async_copy(buf, out_hbm.at[0, pl.ds(0, _PAIR)], sem).wait()


@functools.partial(
    pl.kernel,
    out_type=jax.ShapeDtypeStruct((_NW, _NCHUNK, _RPC, _D), jnp.float32),
    mesh=_sc_mesh,
    scratch_types=[
        pltpu.VMEM((_NCHUNK, _RPC), jnp.int32),
        [pltpu.VMEM((_PAIR, _RPC, _D), jnp.float32) for _ in range(_NSLOT)],
        [[pltpu.SemaphoreType.DMA for _ in range(_PAIR)]
         for _ in range(_NSLOT)],
        [pltpu.SemaphoreType.DMA for _ in range(_NSLOT)],
    ],
)
def _gather(table_hbm, idx_hbm, out_hbm, idx_v, bufs, gsems, wsems):
    wid = lax.axis_index("s") * _NC + lax.axis_index("c")
    pltpu.sync_copy(idx_hbm.at[wid], idx_v)

    def fill(sl, q):
        for k in range(_PAIR):
            pltpu.async_copy(table_hbm.at[idx_v.at[q * _PAIR + k]],
                             bufs[sl].at[k], gsems[sl][k])

    for sl in range(_NSLOT - 1):
        fill(sl, sl)

    def step(q, sl, refill):
        for k in range(_PAIR):
            _wait_gather(table_hbm, bufs[sl].at[k], gsems[sl][k])
        pltpu.async_copy(bufs[sl], out_hbm.at[wid, pl.ds(q * _PAIR, _PAIR)],
                         wsems[sl])
        if refill:
            sl2 = (sl - 1) % _NSLOT
            _wait_write(out_hbm, bufs[sl2], wsems[sl2])
            qr = jnp.minimum(q + _NSLOT - 1, _NPAIR - 1)
            for k in range(_PAIR):
                pltpu.async_copy(table_hbm.at[idx_v.at[qr * _PAIR + k]],
                                 bufs[sl2].at[k], gsems[sl2][k])

    step(0, 0, refill=False)
    fill(_NSLOT - 1, _NSLOT - 1)

    def body(bb, carry):
        q0 = 1 + bb * _NSLOT
        for u in range(_NSLOT):
            step(q0 + u, (1 + u) % _NSLOT, refill=True)
        return carry

    lax.fori_loop(0, (_NPAIR - 1) // _NSLOT, body, 0)

    _wait_write(out_hbm, bufs[(_NPAIR - 1) % _NSLOT],
                wsems[(_NPAIR - 1) % _NSLOT])
    for q in (_NPAIR - 2, _NPAIR - 1):
        sl2 = (q - 1) % _NSLOT
        for k in range(_PAIR):
            _wait_gather(table_hbm, bufs[sl2].at[k], gsems[sl2][k])


def kernel(x, lora_A, lora_B):
    idx = x.astype(jnp.int32).reshape(_NW, _NCHUNK, _RPC)
    table = _build_table(lora_A, lora_B)
    out = _gather(table, idx)
    return out.reshape(_B, _L, _D)
